# Initial kernel scaffold; baseline (speedup 1.0000x reference)
import jax
import jax.numpy as jnp
from jax.experimental import pallas as pl


def kernel(x1, x2, upconv_center_indices, upconv_edge_indices, neigh_orders, W_up, b_up, W1, b1, gamma1, beta1, W2, b2, gamma2, beta2):
    raise NotImplementedError("write your pallas kernel here")


# speedup vs baseline: 2.2539x; 2.2539x over previous
import functools

import jax
import jax.numpy as jnp
from jax import lax
from jax.experimental import pallas as pl
from jax.experimental.pallas import tpu as pltpu
from jax.experimental.pallas import tpu_sc as plsc

_N1 = 10242
_N2 = 4 * _N1 - 6
_NE = 2 * (3 * _N1 - 6)
_F = 128

_NW = 32

_CHUNK = 1344
_N2P = _NW * _CHUNK
_RB = 96
_NB = _CHUNK // _RB

_ECHUNK = _NE // _NW
_ERB = 128
_ENB = _ECHUNK // _ERB

_MB = 512
_N1P = 10752


def _upconv_body(x_ref, w_ref, b_ref, o1_ref, o2_ref):
    y = jnp.dot(x_ref[...], w_ref[...], preferred_element_type=jnp.float32)
    y = y + b_ref[...]
    o1_ref[...] = y[:, : 7 * _F]
    o2_ref[...] = y[:, 7 * _F :]


def _p1_body(u_ref, x2_ref, w_ref, o_ref):
    xc = jnp.concatenate([u_ref[...], x2_ref[...]], axis=1)
    o_ref[...] = jnp.dot(xc, w_ref[...], preferred_element_type=jnp.float32)


def _stats_body(h_ref, gb_ref, ss_ref):
    i = pl.program_id(0)

    @pl.when(i == 0)
    def _init():
        ss_ref[...] = jnp.zeros_like(ss_ref)

    rows = lax.broadcasted_iota(jnp.int32, (_MB, 1), 0) + i * _MB
    blk = jnp.where(rows < _N2, h_ref[...], 0.0)
    ss_ref[0:1, :] += jnp.sum(blk, axis=0, keepdims=True)
    ss_ref[1:2, :] += jnp.sum(blk * blk, axis=0, keepdims=True)

    @pl.when(i == pl.num_programs(0) - 1)
    def _finish():
        inv_n = 1.0 / _N2
        mu = ss_ref[0:1, :] * inv_n
        var = ss_ref[1:2, :] * inv_n - mu * mu
        scale = gb_ref[0:1, :] * lax.rsqrt(var + 1e-5)
        ss_ref[1:2, :] = gb_ref[1:2, :] - mu * scale
        ss_ref[0:1, :] = scale


def _p2_body(h_ref, ss_ref, w_ref, o_ref):
    y = h_ref[...] * ss_ref[0:1, :] + ss_ref[1:2, :]
    y = jnp.where(y >= 0, y, 0.2 * y)
    o_ref[...] = jnp.dot(y, w_ref[...], preferred_element_type=jnp.float32)


def _bnlrelu_body(h_ref, ss_ref, o_ref):
    y = h_ref[...] * ss_ref[0:1, :] + ss_ref[1:2, :]
    o_ref[...] = jnp.where(y >= 0, y, 0.2 * y)


@functools.lru_cache(maxsize=1)
def _sc_fns():
    mesh = plsc.VectorSubcoreMesh(core_axis_name="c", subcore_axis_name="s")

    @functools.partial(
        pl.kernel,
        out_type=jax.ShapeDtypeStruct((_NE // 2, _F), jnp.float32),
        mesh=mesh,
        scratch_types=[
            pltpu.VMEM((_ECHUNK,), jnp.int32),
            pltpu.VMEM((_ERB, _F), jnp.float32),
            pltpu.VMEM((_ERB // 2, _F), jnp.float32),
            pltpu.SemaphoreType.DMA,
        ],
    )
    def edge_gather(pm_hbm, idx_hbm, out_hbm, idx_v, buf_v, st_v, sem):
        wid = lax.axis_index("s") * 2 + lax.axis_index("c")
        pltpu.sync_copy(idx_hbm.at[wid], idx_v)

        def body(j, carry):
            pltpu.async_copy(
                pm_hbm.at[idx_v.at[pl.ds(j * _ERB, _ERB)]], buf_v, sem
            ).wait()

            def pbody(p, pc):
                for c in range(4):
                    sl = pl.ds(c * 16, 16)
                    st_v[p, sl] = buf_v[2 * p, sl]
                    st_v[p, pl.ds(64 + c * 16, 16)] = buf_v[2 * p + 1, sl]
                return pc

            lax.fori_loop(0, _ERB // 2, pbody, 0)
            pltpu.sync_copy(
                st_v,
                out_hbm.at[pl.ds(wid * (_ECHUNK // 2) + j * (_ERB // 2), _ERB // 2)],
            )
            return carry

        lax.fori_loop(0, _ENB, body, 0)

    @functools.partial(
        pl.kernel,
        out_type=jax.ShapeDtypeStruct((_N2P, _F), jnp.float32),
        mesh=mesh,
        scratch_types=[
            pltpu.VMEM((7 * _CHUNK,), jnp.int32),
            pltpu.VMEM((7, _RB, _F), jnp.float32),
            pltpu.VMEM((_RB, _F), jnp.float32),
            pltpu.SemaphoreType.DMA,
        ],
    )
    def gather_acc(tab_hbm, idx_hbm, out_hbm, idx_v, buf_v, acc_v, sem):
        wid = lax.axis_index("s") * 2 + lax.axis_index("c")
        pltpu.sync_copy(idx_hbm.at[wid], idx_v)

        def body(j, carry):
            copies = [
                pltpu.async_copy(
                    tab_hbm.at[idx_v.at[pl.ds(k * _CHUNK + j * _RB, _RB)]],
                    buf_v.at[k],
                    sem,
                )
                for k in range(7)
            ]
            for c in copies:
                c.wait()

            def rbody(r, rcarry):
                for c in range(_F // 16):
                    sl = pl.ds(c * 16, 16)
                    v = buf_v[0, r, sl]
                    for k in range(1, 7):
                        v = v + buf_v[k, r, sl]
                    acc_v[r, sl] = v
                return rcarry

            lax.fori_loop(0, _RB, rbody, 0)
            pltpu.sync_copy(acc_v, out_hbm.at[pl.ds(wid * _CHUNK + j * _RB, _RB)])
            return carry

        lax.fori_loop(0, _NB, body, 0)

    return edge_gather, gather_acc


def _upconv_call(x1p, w_cat_t, b_cat):
    grid = _N1P // _MB
    return pl.pallas_call(
        _upconv_body,
        grid=(grid,),
        in_specs=[
            pl.BlockSpec((_MB, 2 * _F), lambda i: (i, 0)),
            pl.BlockSpec((2 * _F, 14 * _F), lambda i: (0, 0)),
            pl.BlockSpec((1, 14 * _F), lambda i: (0, 0)),
        ],
        out_specs=[
            pl.BlockSpec((_MB, 7 * _F), lambda i: (i, 0)),
            pl.BlockSpec((_MB, 7 * _F), lambda i: (i, 0)),
        ],
        out_shape=[
            jax.ShapeDtypeStruct((_N1P, 7 * _F), jnp.float32),
            jax.ShapeDtypeStruct((_N1P, 7 * _F), jnp.float32),
        ],
    )(x1p, w_cat_t, b_cat)


def _p1_call(up_p, x2p, w1p):
    grid = _N2P // _MB
    return pl.pallas_call(
        _p1_body,
        grid=(grid,),
        in_specs=[
            pl.BlockSpec((_MB, _F), lambda i: (i, 0)),
            pl.BlockSpec((_MB, _F), lambda i: (i, 0)),
            pl.BlockSpec((2 * _F, 7 * _F), lambda i: (0, 0)),
        ],
        out_specs=pl.BlockSpec((_MB, 7 * _F), lambda i: (i, 0)),
        out_shape=jax.ShapeDtypeStruct((_N2P, 7 * _F), jnp.float32),
    )(up_p, x2p, w1p)


def _stats_call(h, gb):
    grid = _N2P // _MB
    return pl.pallas_call(
        _stats_body,
        grid=(grid,),
        in_specs=[
            pl.BlockSpec((_MB, _F), lambda i: (i, 0)),
            pl.BlockSpec((2, _F), lambda i: (0, 0)),
        ],
        out_specs=pl.BlockSpec((8, _F), lambda i: (0, 0)),
        out_shape=jax.ShapeDtypeStruct((8, _F), jnp.float32),
    )(h, gb)


def _p2_call(h, ss, w2p):
    grid = _N2P // _MB
    return pl.pallas_call(
        _p2_body,
        grid=(grid,),
        in_specs=[
            pl.BlockSpec((_MB, _F), lambda i: (i, 0)),
            pl.BlockSpec((8, _F), lambda i: (0, 0)),
            pl.BlockSpec((_F, 7 * _F), lambda i: (0, 0)),
        ],
        out_specs=pl.BlockSpec((_MB, 7 * _F), lambda i: (i, 0)),
        out_shape=jax.ShapeDtypeStruct((_N2P, 7 * _F), jnp.float32),
    )(h, ss, w2p)


def _bnlrelu_call(h, ss):
    grid = _N2P // _MB
    return pl.pallas_call(
        _bnlrelu_body,
        grid=(grid,),
        in_specs=[
            pl.BlockSpec((_MB, _F), lambda i: (i, 0)),
            pl.BlockSpec((8, _F), lambda i: (0, 0)),
        ],
        out_specs=pl.BlockSpec((_MB, _F), lambda i: (i, 0)),
        out_shape=jax.ShapeDtypeStruct((_N2P, _F), jnp.float32),
    )(h, ss)


def kernel(x1, x2, upconv_center_indices, upconv_edge_indices, neigh_orders,
           W_up, b_up, W1, b1, gamma1, beta1, W2, b2, gamma2, beta2):
    f32 = jnp.float32

    w_pm = W_up.reshape(7, 64, 2, 2 * _F).mean(axis=2)
    w_pm = jnp.concatenate([w_pm, jnp.zeros_like(w_pm)], axis=1)
    w_cat_t = jnp.concatenate([W_up, w_pm.reshape(7 * _F, 2 * _F)], axis=0).T
    w1p = W1.reshape(_F, 7, 2 * _F).transpose(2, 1, 0).reshape(2 * _F, 7 * _F)
    w2p = W2.reshape(_F, 7, _F).transpose(2, 1, 0).reshape(_F, 7 * _F)
    gb1 = jnp.stack([gamma1, beta1]).astype(f32)
    gb2 = jnp.stack([gamma2, beta2]).astype(f32)
    b_pm = b_up.reshape(7, 64, 2).mean(axis=2)
    b_pm = jnp.concatenate([b_pm, jnp.zeros_like(b_pm)], axis=1)
    b_cat = jnp.concatenate([b_up, b_pm.reshape(7 * _F)])[None, :]

    x1p = jnp.pad(x1, ((0, _N1P - _N1), (0, 0)))
    up_p, pm_p = _upconv_call(x1p, w_cat_t, b_cat)
    up_flat = up_p[:_N1].reshape(_N1 * 7, _F)
    pm_flat = pm_p[:_N1].reshape(_N1 * 7, _F)

    edge_gather, gather_acc = _sc_fns()

    eidx = upconv_edge_indices.reshape(_NW, _ECHUNK)
    u2m = edge_gather(pm_flat, eidx)

    u1 = up_flat[:_N1]
    up_out = jnp.concatenate([u1, u2m], axis=0)
    up_out_p = jnp.pad(up_out, ((0, _N2P - _N2), (0, 0)))
    x2p = jnp.pad(x2, ((0, _N2P - _N2), (0, 0)))

    ng2 = neigh_orders.reshape(_N2, 7) * 7 + jnp.arange(7, dtype=jnp.int32)[None, :]
    ngt = jnp.pad(ng2.T, ((0, 0), (0, _N2P - _N2)))
    nidx = ngt.reshape(7, _NW, _CHUNK).transpose(1, 0, 2).reshape(_NW, 7 * _CHUNK)

    p1 = _p1_call(up_out_p, x2p, w1p)
    h1 = gather_acc(p1.reshape(_N2P * 7, _F), nidx)
    ss1 = _stats_call(h1, gb1)

    p2 = _p2_call(h1, ss1, w2p)
    h2 = gather_acc(p2.reshape(_N2P * 7, _F), nidx)
    ss2 = _stats_call(h2, gb2)

    out = _bnlrelu_call(h2, ss2)
    return out[:_N2]
